# baseline (speedup 1.0000x reference)
import functools

import jax
import jax.numpy as jnp
import numpy as np
from jax import lax
from jax.experimental import pallas as pl
from jax.experimental.pallas import tpu as pltpu
from jax.experimental.pallas import tpu_sc as plsc

NC = 2
NS = 16
CHUNK = 128
NBUF = 2
NPHASE = 2


def _sc_scatter(src_p, dst_p, features, *, n_pad, n_chunks):
    feat_rows, D = features.shape
    rpt = n_pad // NS
    pchunks = n_chunks // NPHASE
    n_outer = pchunks // NBUF
    mesh = plsc.VectorSubcoreMesh(core_axis_name="c", subcore_axis_name="s")
    last_rows = max(0, feat_rows - (NS - 1) * rpt)

    @functools.partial(
        pl.kernel,
        mesh=mesh,
        out_type=jax.ShapeDtypeStruct((NC, n_pad, D), jnp.float32),
        scratch_types=[
            pltpu.VMEM_SHARED((n_pad, D), jnp.float32),
            pltpu.VMEM((pchunks, CHUNK), jnp.int32),
            pltpu.VMEM((pchunks, CHUNK), jnp.int32),
            pltpu.VMEM((NBUF, CHUNK, D), jnp.float32),
            pltpu.SemaphoreType.DMA,
            pltpu.SemaphoreType.DMA,
        ],
    )
    def sc_kernel(src_hbm, dst_hbm, feat_hbm, part_hbm,
                  agg, srcs_v, dsts_v, rows_v, *gsems):
        c = lax.axis_index("c")
        s = lax.axis_index("s")
        wid = s * NC + c
        row0 = pl.multiple_of(s * rpt, 8)

        for p in range(NPHASE):
            cbase = pl.multiple_of(wid * n_chunks + p * pchunks, 8)
            pltpu.sync_copy(src_hbm.at[pl.ds(cbase, pchunks)], srcs_v)
            pltpu.sync_copy(dst_hbm.at[pl.ds(cbase, pchunks)], dsts_v)

            for b in range(NBUF):
                pltpu.async_copy(feat_hbm.at[srcs_v.at[b]], rows_v.at[b], gsems[b])

            if p == 0:
                @pl.when(s < NS - 1)
                def _():
                    pltpu.sync_copy(feat_hbm.at[pl.ds(row0, rpt)],
                                    agg.at[pl.ds(row0, rpt)])

                if last_rows > 0:
                    @pl.when(s == NS - 1)
                    def _():
                        pltpu.sync_copy(feat_hbm.at[pl.ds(row0, last_rows)],
                                        agg.at[pl.ds(row0, last_rows)])
                plsc.subcore_barrier()

            def outer(o, carry):
                for b in range(NBUF):
                    i = o * NBUF + b
                    pltpu.make_async_copy(
                        feat_hbm.at[srcs_v.at[i]], rows_v.at[b], gsems[b]
                    ).wait()
                    pltpu.sync_copy(rows_v.at[b], agg.at[dsts_v.at[i]], add=True)

                    @pl.when(i + NBUF < pchunks)
                    def _():
                        pltpu.async_copy(
                            feat_hbm.at[srcs_v.at[i + NBUF]], rows_v.at[b], gsems[b]
                        )
                return carry

            lax.fori_loop(0, n_outer, outer, 0)
        plsc.subcore_barrier()

        pltpu.sync_copy(agg.at[pl.ds(row0, rpt)], part_hbm.at[c, pl.ds(row0, rpt)])

    return sc_kernel(src_p, dst_p, features)


def _tc_matmul_body(p0_ref, p1_ref, f_ref, w_ref, b_ref, o_ref):
    agg = p0_ref[0] + p1_ref[0] - f_ref[...]
    o_ref[...] = (
        jnp.dot(agg, w_ref[...], preferred_element_type=jnp.float32) + b_ref[...]
    )


def kernel(features, edge_index, weight, bias):
    N, D = features.shape
    E = edge_index.shape[1]

    quantum = NPHASE * 8
    n_chunks = -(-E // (NC * NS * CHUNK * quantum)) * quantum
    E_pad = n_chunks * NC * NS * CHUNK
    n_pad = -(-(N + 1) // (NS * 8)) * NS * 8

    src = edge_index[0]
    dst = edge_index[1]
    pad = E_pad - E
    pad_iota = np.arange(pad, dtype=np.int32)
    src_tail = jnp.asarray(pad_iota % N)
    dst_tail = jnp.asarray(N + pad_iota % (n_pad - N))
    src_p = jnp.concatenate([src, src_tail]).reshape(E_pad // CHUNK, CHUNK)
    dst_p = jnp.concatenate([dst, dst_tail]).reshape(E_pad // CHUNK, CHUNK)
    parts = _sc_scatter(src_p, dst_p, features,
                        n_pad=n_pad, n_chunks=n_chunks)

    BM = 2048
    grid = -(-N // BM)
    h = pl.pallas_call(
        _tc_matmul_body,
        grid=(grid,),
        in_specs=[
            pl.BlockSpec((1, BM, D), lambda i: (0, i, 0)),
            pl.BlockSpec((1, BM, D), lambda i: (1, i, 0)),
            pl.BlockSpec((BM, D), lambda i: (i, 0)),
            pl.BlockSpec((D, D), lambda i: (0, 0)),
            pl.BlockSpec((1, D), lambda i: (0, 0)),
        ],
        out_specs=pl.BlockSpec((BM, D), lambda i: (i, 0)),
        out_shape=jax.ShapeDtypeStruct((N, D), jnp.float32),
    )(parts, parts, features, weight, bias.reshape(1, D))
    return h


# speedup vs baseline: 1.0686x; 1.0686x over previous
import functools

import jax
import jax.numpy as jnp
import numpy as np
from jax import lax
from jax.experimental import pallas as pl
from jax.experimental.pallas import tpu as pltpu
from jax.experimental.pallas import tpu_sc as plsc

NC = 2
NS = 16
CHUNK = 128
NBUF = 2
NPHASE = 2


def _sc_scatter(ei_p, features, *, n_pad, n_chunks):
    feat_rows, D = features.shape
    rpt = n_pad // NS
    pchunks = n_chunks // NPHASE
    n_outer = pchunks // NBUF
    mesh = plsc.VectorSubcoreMesh(core_axis_name="c", subcore_axis_name="s")
    last_rows = max(0, feat_rows - (NS - 1) * rpt)

    @functools.partial(
        pl.kernel,
        mesh=mesh,
        out_type=jax.ShapeDtypeStruct((NC, n_pad, D), jnp.float32),
        scratch_types=[
            pltpu.VMEM_SHARED((n_pad, D), jnp.float32),
            pltpu.VMEM((pchunks, CHUNK), jnp.int32),
            pltpu.VMEM((pchunks, CHUNK), jnp.int32),
            pltpu.VMEM((NBUF, CHUNK, D), jnp.float32),
            pltpu.SemaphoreType.DMA,
            pltpu.SemaphoreType.DMA,
        ],
    )
    def sc_kernel(ei_hbm, feat_hbm, part_hbm,
                  agg, srcs_v, dsts_v, rows_v, *gsems):
        c = lax.axis_index("c")
        s = lax.axis_index("s")
        wid = s * NC + c
        row0 = pl.multiple_of(s * rpt, 8)

        for p in range(NPHASE):
            cbase = pl.multiple_of(wid * n_chunks + p * pchunks, 8)
            pltpu.sync_copy(ei_hbm.at[0, pl.ds(cbase, pchunks)], srcs_v)
            pltpu.sync_copy(ei_hbm.at[1, pl.ds(cbase, pchunks)], dsts_v)

            for b in range(NBUF):
                pltpu.async_copy(feat_hbm.at[srcs_v.at[b]], rows_v.at[b], gsems[b])

            if p == 0:
                @pl.when(s < NS - 1)
                def _():
                    pltpu.sync_copy(feat_hbm.at[pl.ds(row0, rpt)],
                                    agg.at[pl.ds(row0, rpt)])

                if last_rows > 0:
                    @pl.when(s == NS - 1)
                    def _():
                        pltpu.sync_copy(feat_hbm.at[pl.ds(row0, last_rows)],
                                        agg.at[pl.ds(row0, last_rows)])
                plsc.subcore_barrier()

            def outer(o, carry):
                for b in range(NBUF):
                    i = o * NBUF + b
                    pltpu.make_async_copy(
                        feat_hbm.at[srcs_v.at[i]], rows_v.at[b], gsems[b]
                    ).wait()
                    pltpu.sync_copy(rows_v.at[b], agg.at[dsts_v.at[i]], add=True)

                    @pl.when(i + NBUF < pchunks)
                    def _():
                        pltpu.async_copy(
                            feat_hbm.at[srcs_v.at[i + NBUF]], rows_v.at[b], gsems[b]
                        )
                return carry

            lax.fori_loop(0, n_outer, outer, 0)
        plsc.subcore_barrier()

        pltpu.sync_copy(agg.at[pl.ds(row0, rpt)], part_hbm.at[c, pl.ds(row0, rpt)])

    return sc_kernel(ei_p, features)


def _tc_matmul_body(p0_ref, p1_ref, f_ref, w_ref, b_ref, o_ref):
    agg = p0_ref[0] + p1_ref[0] - f_ref[...]
    o_ref[...] = (
        jnp.dot(agg, w_ref[...], preferred_element_type=jnp.float32) + b_ref[...]
    )


def kernel(features, edge_index, weight, bias):
    N, D = features.shape
    E = edge_index.shape[1]

    quantum = NPHASE * 8
    n_chunks = -(-E // (NC * NS * CHUNK * quantum)) * quantum
    E_pad = n_chunks * NC * NS * CHUNK
    n_pad = -(-(N + 1) // (NS * 8)) * NS * 8

    pad = E_pad - E
    pad_iota = np.arange(pad, dtype=np.int32)
    tails = jnp.asarray(
        np.stack([pad_iota % N, N + pad_iota % (n_pad - N)]))
    ei_p = jnp.concatenate([edge_index, tails], axis=1).reshape(
        2, E_pad // CHUNK, CHUNK)
    parts = _sc_scatter(ei_p, features, n_pad=n_pad, n_chunks=n_chunks)

    BM = 2048
    grid = -(-N // BM)
    h = pl.pallas_call(
        _tc_matmul_body,
        grid=(grid,),
        in_specs=[
            pl.BlockSpec((1, BM, D), lambda i: (0, i, 0)),
            pl.BlockSpec((1, BM, D), lambda i: (1, i, 0)),
            pl.BlockSpec((BM, D), lambda i: (i, 0)),
            pl.BlockSpec((D, D), lambda i: (0, 0)),
            pl.BlockSpec((1, D), lambda i: (0, 0)),
        ],
        out_specs=pl.BlockSpec((BM, D), lambda i: (i, 0)),
        out_shape=jax.ShapeDtypeStruct((N, D), jnp.float32),
    )(parts, parts, features, weight, bias.reshape(1, D))
    return h


# speedup vs baseline: 1.0751x; 1.0060x over previous
import functools

import jax
import jax.numpy as jnp
import numpy as np
from jax import lax
from jax.experimental import pallas as pl
from jax.experimental.pallas import tpu as pltpu
from jax.experimental.pallas import tpu_sc as plsc

NC = 2
NS = 16
CHUNK = 128
NBUF = 2
NPHASE = 2


def _sc_scatter(ei_p, features, *, n_pad, n_chunks):
    feat_rows, D = features.shape
    rpt = n_pad // NS
    pchunks = n_chunks // NPHASE
    n_outer = pchunks // NBUF
    mesh = plsc.VectorSubcoreMesh(core_axis_name="c", subcore_axis_name="s")
    last_rows = max(0, feat_rows - (NS - 1) * rpt)

    @functools.partial(
        pl.kernel,
        mesh=mesh,
        out_type=jax.ShapeDtypeStruct((NC, n_pad, D), jnp.float32),
        scratch_types=[
            pltpu.VMEM_SHARED((n_pad, D), jnp.float32),
            pltpu.VMEM((pchunks, CHUNK), jnp.int32),
            pltpu.VMEM((pchunks, CHUNK), jnp.int32),
            pltpu.VMEM((NBUF, CHUNK, D), jnp.float32),
            pltpu.SemaphoreType.DMA,
            pltpu.SemaphoreType.DMA,
        ],
    )
    def sc_kernel(ei_hbm, feat_hbm, part_hbm,
                  agg, srcs_v, dsts_v, rows_v, *gsems):
        c = lax.axis_index("c")
        s = lax.axis_index("s")
        wid = s * NC + c
        row0 = pl.multiple_of(s * rpt, 8)

        for p in range(NPHASE):
            cbase = pl.multiple_of(wid * n_chunks + p * pchunks, 8)
            pltpu.sync_copy(ei_hbm.at[0, pl.ds(cbase, pchunks)], srcs_v)
            pltpu.sync_copy(ei_hbm.at[1, pl.ds(cbase, pchunks)], dsts_v)

            for b in range(NBUF):
                pltpu.async_copy(feat_hbm.at[srcs_v.at[b]], rows_v.at[b], gsems[b])

            if p == 0:
                @pl.when(s < NS - 1)
                def _():
                    pltpu.sync_copy(feat_hbm.at[pl.ds(row0, rpt)],
                                    agg.at[pl.ds(row0, rpt)])

                if last_rows > 0:
                    @pl.when(s == NS - 1)
                    def _():
                        pltpu.sync_copy(feat_hbm.at[pl.ds(row0, last_rows)],
                                        agg.at[pl.ds(row0, last_rows)])
                plsc.subcore_barrier()

            def outer(o, carry):
                for b in range(NBUF):
                    i = o * NBUF + b
                    pltpu.make_async_copy(
                        feat_hbm.at[srcs_v.at[i]], rows_v.at[b], gsems[b]
                    ).wait()
                    pltpu.sync_copy(rows_v.at[b], agg.at[dsts_v.at[i]], add=True)

                    @pl.when(i + NBUF < pchunks)
                    def _():
                        pltpu.async_copy(
                            feat_hbm.at[srcs_v.at[i + NBUF]], rows_v.at[b], gsems[b]
                        )
                return carry

            lax.fori_loop(0, n_outer, outer, 0)
        plsc.subcore_barrier()

        pltpu.sync_copy(agg.at[pl.ds(row0, rpt)], part_hbm.at[c, pl.ds(row0, rpt)])

    return sc_kernel(ei_p, features)


def _tc_matmul_body(p0_ref, p1_ref, f_ref, w_ref, b_ref, o_ref):
    agg = p0_ref[0] + p1_ref[0] - f_ref[...]
    o_ref[...] = (
        jnp.dot(agg, w_ref[...], preferred_element_type=jnp.float32) + b_ref[...]
    )


def kernel(features, edge_index, weight, bias):
    N, D = features.shape
    E = edge_index.shape[1]

    quantum = NPHASE * 8
    n_chunks = -(-E // (NC * NS * CHUNK * quantum)) * quantum
    E_pad = n_chunks * NC * NS * CHUNK
    n_pad = -(-(N + 1) // (NS * 8)) * NS * 8

    pad = E_pad - E
    pad_iota = np.arange(pad, dtype=np.int32)
    tails = jnp.asarray(
        np.stack([pad_iota % N, N + pad_iota % (n_pad - N)]))
    ei_p = jnp.concatenate([edge_index, tails], axis=1).reshape(
        2, E_pad // CHUNK, CHUNK)
    parts = _sc_scatter(ei_p, features, n_pad=n_pad, n_chunks=n_chunks)

    BM = 3392
    grid = -(-N // BM)
    h = pl.pallas_call(
        _tc_matmul_body,
        grid=(grid,),
        in_specs=[
            pl.BlockSpec((1, BM, D), lambda i: (0, i, 0)),
            pl.BlockSpec((1, BM, D), lambda i: (1, i, 0)),
            pl.BlockSpec((BM, D), lambda i: (i, 0)),
            pl.BlockSpec((D, D), lambda i: (0, 0)),
            pl.BlockSpec((1, D), lambda i: (0, 0)),
        ],
        out_specs=pl.BlockSpec((BM, D), lambda i: (i, 0)),
        out_shape=jax.ShapeDtypeStruct((N, D), jnp.float32),
    )(parts, parts, features, weight, bias.reshape(1, D))
    return h
